# Initial kernel scaffold; baseline (speedup 1.0000x reference)
import jax
import jax.numpy as jnp
from jax.experimental import pallas as pl


def kernel(x, edge_index, edge_attr, batch, W1, b1, W2, b2, W3, b3, LW1, Lb1, LW2, Lb2, LW3, Lb3):
    raise NotImplementedError("write your pallas kernel here")


# speedup vs baseline: 9.1592x; 9.1592x over previous
import functools

import jax
import jax.numpy as jnp
from jax import lax
from jax.experimental import pallas as pl
from jax.experimental.pallas import tpu as pltpu
from jax.experimental.pallas import tpu_sc as plsc

_N = 10000
_NP = 10240
_E = 320000
_EP = 327680
_G = 128
_NSUB = 16
_NCORE = 2
_ROWS = _EP // 128
_CPT = _ROWS // 32
_CPT2 = _ROWS // 16
_SEG = _NP // _NSUB

_mesh = plsc.VectorSubcoreMesh(core_axis_name="c", subcore_axis_name="s")
_sc_params = pltpu.CompilerParams(needs_layout_passes=False,
                                  use_tc_tiling_on_sc=False)


def _splat(v16, j):
    return v16.at[jnp.full((16,), j, jnp.int32)].get(mode="promise_in_bounds")


@functools.partial(
    pl.kernel,
    out_type=jax.ShapeDtypeStruct((_NCORE * _NP,), jnp.float32),
    mesh=_mesh,
    compiler_params=_sc_params,
    scratch_types=[
        pltpu.VMEM((_CPT, 128), jnp.int32),
        pltpu.VMEM((_CPT, 128), jnp.float32),
        pltpu.VMEM((_SEG,), jnp.float32),
        pltpu.VMEM_SHARED((_NP,), jnp.float32),
    ],
)
def _deg_sc(col_h, ew_h, deg_h, col_v, ew_v, zv, acc_sh):
    c = lax.axis_index("c")
    s = lax.axis_index("s")
    w = c * _NSUB + s
    pltpu.sync_copy(col_h.at[pl.ds(w * _CPT, _CPT)], col_v)
    pltpu.sync_copy(ew_h.at[pl.ds(w * _CPT, _CPT)], ew_v)

    def zb(i, carry):
        zv[pl.ds(i * 16, 16)] = jnp.zeros((16,), jnp.float32)
        return carry

    lax.fori_loop(0, _SEG // 16, zb, 0)
    pltpu.sync_copy(zv, acc_sh.at[pl.ds(s * _SEG, _SEG)])
    plsc.subcore_barrier()

    def body(k, carry):
        pltpu.sync_copy(ew_v.at[k], acc_sh.at[col_v.at[k]], add=True)
        return carry

    lax.fori_loop(0, _CPT, body, 0)
    plsc.subcore_barrier()
    pltpu.sync_copy(
        acc_sh.at[pl.ds(s * _SEG, _SEG)],
        deg_h.at[pl.ds(c * _NP + s * _SEG, _SEG)],
    )


@functools.partial(
    pl.kernel,
    out_type=(
        jax.ShapeDtypeStruct((_NCORE * _NP, 16), jnp.float32),
        jax.ShapeDtypeStruct((_ROWS, 128), jnp.float32),
    ),
    mesh=_mesh,
    compiler_params=_sc_params,
    scratch_types=[
        pltpu.VMEM((_CPT, 128), jnp.int32),
        pltpu.VMEM((_CPT, 128), jnp.int32),
        pltpu.VMEM((_CPT, 128), jnp.float32),
        pltpu.VMEM((_CPT, 128), jnp.float32),
        pltpu.VMEM((_NP,), jnp.float32),
        pltpu.VMEM((128, 16), jnp.float32),
        pltpu.VMEM((128, 16), jnp.float32),
        pltpu.VMEM_SHARED((_NP, 16), jnp.float32),
        pltpu.SemaphoreType.DMA,
    ],
)
def _spmm1_sc(h_h, row_h, col_h, ew_h, dinv_h, out_h, norm_out_h,
              row_v, col_v, ew_v, norm_v, dinv_v, gbuf, zbuf, acc_sh, sem):
    c = lax.axis_index("c")
    s = lax.axis_index("s")
    w = c * _NSUB + s
    pltpu.sync_copy(row_h.at[pl.ds(w * _CPT, _CPT)], row_v)
    pltpu.sync_copy(col_h.at[pl.ds(w * _CPT, _CPT)], col_v)
    pltpu.sync_copy(ew_h.at[pl.ds(w * _CPT, _CPT)], ew_v)
    pltpu.sync_copy(dinv_h, dinv_v)

    def zb(i, carry):
        zbuf[i, :] = jnp.zeros((16,), jnp.float32)
        return carry

    lax.fori_loop(0, 128, zb, 0)

    def za(i, carry):
        pltpu.sync_copy(zbuf, acc_sh.at[pl.ds(s * _SEG + i * 128, 128)])
        return carry

    lax.fori_loop(0, _SEG // 128, za, 0)

    def nb(k, carry):
        for j16 in range(8):
            sl = pl.ds(j16 * 16, 16)
            r16 = row_v[k, sl]
            c16 = col_v[k, sl]
            nrm = (plsc.load_gather(dinv_v, [r16]) * ew_v[k, sl]
                   * plsc.load_gather(dinv_v, [c16]))
            norm_v[k, sl] = nrm
        return carry

    lax.fori_loop(0, _CPT, nb, 0)
    pltpu.sync_copy(norm_v, norm_out_h.at[pl.ds(w * _CPT, _CPT)])
    plsc.subcore_barrier()

    def body(k, carry):
        pltpu.async_copy(h_h.at[row_v.at[k]], gbuf, sem).wait()
        for j16 in range(8):
            nrm16 = norm_v[k, pl.ds(j16 * 16, 16)]
            for j in range(16):
                e = j16 * 16 + j
                gbuf[e, :] = gbuf[e, :] * _splat(nrm16, j)
        pltpu.sync_copy(gbuf, acc_sh.at[col_v.at[k]], add=True)
        return carry

    lax.fori_loop(0, _CPT, body, 0)
    plsc.subcore_barrier()
    pltpu.sync_copy(
        acc_sh.at[pl.ds(s * _SEG, _SEG)],
        out_h.at[pl.ds(c * _NP + s * _SEG, _SEG)],
    )


@functools.partial(
    pl.kernel,
    out_type=jax.ShapeDtypeStruct((_NCORE * _NP, 64), jnp.float32),
    mesh=_mesh,
    compiler_params=_sc_params,
    scratch_types=[
        pltpu.VMEM((_CPT, 128), jnp.int32),
        pltpu.VMEM((_CPT, 128), jnp.int32),
        pltpu.VMEM((_CPT, 128), jnp.float32),
        pltpu.VMEM((128, 64), jnp.float32),
        pltpu.VMEM((128, 64), jnp.float32),
        pltpu.VMEM_SHARED((_NP, 64), jnp.float32),
        pltpu.SemaphoreType.DMA,
    ],
)
def _spmm2_sc(h_h, row_h, col_h, norm_h, out_h,
              row_v, col_v, norm_v, gbuf, zbuf, acc_sh, sem):
    c = lax.axis_index("c")
    s = lax.axis_index("s")
    w = c * _NSUB + s
    pltpu.sync_copy(row_h.at[pl.ds(w * _CPT, _CPT)], row_v)
    pltpu.sync_copy(col_h.at[pl.ds(w * _CPT, _CPT)], col_v)
    pltpu.sync_copy(norm_h.at[pl.ds(w * _CPT, _CPT)], norm_v)

    def zb(i, carry):
        for v in range(4):
            zbuf[i, pl.ds(v * 16, 16)] = jnp.zeros((16,), jnp.float32)
        return carry

    lax.fori_loop(0, 128, zb, 0)

    def za(i, carry):
        pltpu.sync_copy(zbuf, acc_sh.at[pl.ds(s * _SEG + i * 128, 128)])
        return carry

    lax.fori_loop(0, _SEG // 128, za, 0)
    plsc.subcore_barrier()

    def body(k, carry):
        pltpu.async_copy(h_h.at[row_v.at[k]], gbuf, sem).wait()
        for j16 in range(8):
            nrm16 = norm_v[k, pl.ds(j16 * 16, 16)]
            for j in range(16):
                e = j16 * 16 + j
                spl = _splat(nrm16, j)
                for v in range(4):
                    sl = pl.ds(v * 16, 16)
                    gbuf[e, sl] = gbuf[e, sl] * spl
        pltpu.sync_copy(gbuf, acc_sh.at[col_v.at[k]], add=True)
        return carry

    lax.fori_loop(0, _CPT, body, 0)
    plsc.subcore_barrier()
    pltpu.sync_copy(
        acc_sh.at[pl.ds(s * _SEG, _SEG)],
        out_h.at[pl.ds(c * _NP + s * _SEG, _SEG)],
    )


@functools.partial(
    pl.kernel,
    out_type=jax.ShapeDtypeStruct((_NCORE * _NP, 128), jnp.float32),
    mesh=_mesh,
    compiler_params=_sc_params,
    scratch_types=[
        pltpu.VMEM((16, 128), jnp.int32),
        pltpu.VMEM((16, 128), jnp.int32),
        pltpu.VMEM((16, 128), jnp.float32),
        pltpu.VMEM((128, 128), jnp.float32),
        pltpu.VMEM((128, 128), jnp.float32),
        pltpu.VMEM_SHARED((_NP, 128), jnp.float32),
        pltpu.SemaphoreType.DMA,
    ],
)
def _spmm3_sc(h_h, row_h, col_h, norm_h, out_h,
              row_v, col_v, norm_v, gbuf, zbuf, acc_sh, sem):
    c = lax.axis_index("c")
    s = lax.axis_index("s")
    off16 = jnp.full((16,), c * _NP, jnp.int32)

    def zb(i, carry):
        for v in range(8):
            zbuf[i, pl.ds(v * 16, 16)] = jnp.zeros((16,), jnp.float32)
        return carry

    lax.fori_loop(0, 128, zb, 0)

    def za(i, carry):
        pltpu.sync_copy(zbuf, acc_sh.at[pl.ds(s * _SEG + i * 128, 128)])
        return carry

    lax.fori_loop(0, _SEG // 128, za, 0)
    plsc.subcore_barrier()

    def slab(m, carry):
        base = s * _CPT2 + m * 16
        pltpu.sync_copy(row_h.at[pl.ds(base, 16)], row_v)
        pltpu.sync_copy(col_h.at[pl.ds(base, 16)], col_v)
        pltpu.sync_copy(norm_h.at[pl.ds(base, 16)], norm_v)

        def ob(k, carry2):
            for j16 in range(8):
                sl = pl.ds(j16 * 16, 16)
                row_v[k, sl] = row_v[k, sl] + off16
            return carry2

        lax.fori_loop(0, 16, ob, 0)

        def body(k, carry2):
            pltpu.async_copy(h_h.at[row_v.at[k]], gbuf, sem).wait()
            for j16 in range(8):
                nrm16 = norm_v[k, pl.ds(j16 * 16, 16)]
                for j in range(16):
                    e = j16 * 16 + j
                    spl = _splat(nrm16, j)
                    for v in range(8):
                        sl = pl.ds(v * 16, 16)
                        gbuf[e, sl] = gbuf[e, sl] * spl
            pltpu.sync_copy(gbuf, acc_sh.at[col_v.at[k]], add=True)
            return carry2

        lax.fori_loop(0, 16, body, 0)
        return carry

    lax.fori_loop(0, _CPT2 // 16, slab, 0)
    plsc.subcore_barrier()
    pltpu.sync_copy(
        acc_sh.at[pl.ds(s * _SEG, _SEG)],
        out_h.at[pl.ds(c * _NP + s * _SEG, _SEG)],
    )


_BT = 1024
_NPB = _NP // _BT


def _tca_body(d0, d1, xb, w1, dinv_o, dinv2_o, h1_o):
    deg = d0[...] + d1[...] + 1.0
    dinv_o[...] = lax.rsqrt(deg)
    dinv2_o[...] = 1.0 / deg
    h1_o[...] = jnp.dot(xb[...], w1[...], preferred_element_type=jnp.float32)


def _tc_a(d0, d1, x_p, W1):
    return pl.pallas_call(
        _tca_body,
        grid=(_NPB,),
        in_specs=[
            pl.BlockSpec((_BT, 1), lambda i: (i, 0)),
            pl.BlockSpec((_BT, 1), lambda i: (i, 0)),
            pl.BlockSpec((_BT, 128), lambda i: (i, 0)),
            pl.BlockSpec((128, 16), lambda i: (0, 0)),
        ],
        out_specs=[
            pl.BlockSpec((_BT, 1), lambda i: (i, 0)),
            pl.BlockSpec((_BT, 1), lambda i: (i, 0)),
            pl.BlockSpec((_BT, 16), lambda i: (i, 0)),
        ],
        out_shape=[
            jax.ShapeDtypeStruct((_NP, 1), jnp.float32),
            jax.ShapeDtypeStruct((_NP, 1), jnp.float32),
            jax.ShapeDtypeStruct((_NP, 16), jnp.float32),
        ],
    )(d0, d1, x_p, W1)


def _tcb_body(oa, ob_, hb, dinv2, bias, w, out_o):
    pre = oa[...] + ob_[...] + dinv2[...] * hb[...] + bias[...]
    act = jax.nn.silu(pre)
    out_o[...] = jnp.dot(act, w[...], preferred_element_type=jnp.float32)


def _tc_layer(outp, h, dinv2, bias, w, fin, fout):
    return pl.pallas_call(
        _tcb_body,
        grid=(_NPB,),
        in_specs=[
            pl.BlockSpec((_BT, fin), lambda i: (i, 0)),
            pl.BlockSpec((_BT, fin), lambda i: (_NPB + i, 0)),
            pl.BlockSpec((_BT, fin), lambda i: (i, 0)),
            pl.BlockSpec((_BT, 1), lambda i: (i, 0)),
            pl.BlockSpec((1, fin), lambda i: (0, 0)),
            pl.BlockSpec((fin, fout), lambda i: (0, 0)),
        ],
        out_specs=pl.BlockSpec((_BT, fout), lambda i: (i, 0)),
        out_shape=jax.ShapeDtypeStruct((_NP, fout), jnp.float32),
    )(outp, outp, h, dinv2, bias, w)


def _tcc_body(oa, ob_, hb, dinv2, bias, w, out_o):
    pre = oa[...] + ob_[...] + dinv2[...] * hb[...] + bias[...]
    act = jax.nn.silu(pre)
    out_o[...] = jnp.dot(act, w[...], preferred_element_type=jnp.float32)


def _tc_layer3(outp, h2, dinv2, b2, W3):
    return pl.pallas_call(
        _tcc_body,
        grid=(_NPB, 2),
        in_specs=[
            pl.BlockSpec((_BT, 64), lambda i, j: (i, 0)),
            pl.BlockSpec((_BT, 64), lambda i, j: (_NPB + i, 0)),
            pl.BlockSpec((_BT, 64), lambda i, j: (i, 0)),
            pl.BlockSpec((_BT, 1), lambda i, j: (i, 0)),
            pl.BlockSpec((1, 64), lambda i, j: (0, 0)),
            pl.BlockSpec((64, 128), lambda i, j: (0, j)),
        ],
        out_specs=pl.BlockSpec((_BT, 128), lambda i, j: (j * _NPB + i, 0)),
        out_shape=jax.ShapeDtypeStruct((2 * _NP, 128), jnp.float32),
    )(outp, outp, h2, dinv2, b2, W3)


def _tcd_body(o3a, o3b, h3a, h3b, dinv2, b3, batchb,
              lw1, lb1, lw2, lb2, lw3, lb3, out_o, sumsA, sumsB, cnts):
    i = pl.program_id(0)

    @pl.when(i == 0)
    def _():
        sumsA[...] = jnp.zeros_like(sumsA)
        sumsB[...] = jnp.zeros_like(sumsB)
        cnts[...] = jnp.zeros_like(cnts)

    d2 = dinv2[...]
    x4a = jax.nn.silu(o3a[...] + d2 * h3a[...] + b3[..., :128])
    x4b = jax.nn.silu(o3b[...] + d2 * h3b[...] + b3[..., 128:])
    ids = lax.broadcasted_iota(jnp.int32, (_G, _BT), 0)
    P = (batchb[...] == ids).astype(jnp.float32)
    sumsA[...] += jnp.dot(P, x4a, preferred_element_type=jnp.float32)
    sumsB[...] += jnp.dot(P, x4b, preferred_element_type=jnp.float32)
    cnts[...] += jnp.sum(P, axis=1, keepdims=True)

    @pl.when(i == _NPB - 1)
    def _():
        inv = 1.0 / jnp.maximum(cnts[...], 1.0)
        g = jnp.concatenate([sumsA[...], sumsB[...]], axis=1) * inv
        g = jax.nn.silu(jnp.dot(g, lw1[...], preferred_element_type=jnp.float32) + lb1[...])
        g = jax.nn.silu(jnp.dot(g, lw2[...], preferred_element_type=jnp.float32) + lb2[...])
        out_o[...] = jnp.dot(g, lw3[...], preferred_element_type=jnp.float32) + lb3[...]


def _tc_d(out3, h3f, dinv2, b3, batch2d, LW1, Lb1, LW2, Lb2, LW3, Lb3):
    return pl.pallas_call(
        _tcd_body,
        grid=(_NPB,),
        in_specs=[
            pl.BlockSpec((_BT, 128), lambda i: (i, 0)),
            pl.BlockSpec((_BT, 128), lambda i: (_NPB + i, 0)),
            pl.BlockSpec((_BT, 128), lambda i: (i, 0)),
            pl.BlockSpec((_BT, 128), lambda i: (_NPB + i, 0)),
            pl.BlockSpec((_BT, 1), lambda i: (i, 0)),
            pl.BlockSpec((1, 256), lambda i: (0, 0)),
            pl.BlockSpec((1, _BT), lambda i: (0, i)),
            pl.BlockSpec((256, 128), lambda i: (0, 0)),
            pl.BlockSpec((1, 128), lambda i: (0, 0)),
            pl.BlockSpec((128, 64), lambda i: (0, 0)),
            pl.BlockSpec((1, 64), lambda i: (0, 0)),
            pl.BlockSpec((64, 1), lambda i: (0, 0)),
            pl.BlockSpec((1, 1), lambda i: (0, 0)),
        ],
        out_specs=pl.BlockSpec((_G, 1), lambda i: (0, 0)),
        out_shape=jax.ShapeDtypeStruct((_G, 1), jnp.float32),
        scratch_shapes=[
            pltpu.VMEM((_G, 128), jnp.float32),
            pltpu.VMEM((_G, 128), jnp.float32),
            pltpu.VMEM((_G, 1), jnp.float32),
        ],
    )(out3, out3, h3f, h3f, dinv2, b3, batch2d,
      LW1, Lb1, LW2, Lb2, LW3, Lb3)


def kernel(x, edge_index, edge_attr, batch, W1, b1, W2, b2, W3, b3,
           LW1, Lb1, LW2, Lb2, LW3, Lb3):
    row = edge_index[0].astype(jnp.int32)
    col = edge_index[1].astype(jnp.int32)
    ew = edge_attr.astype(jnp.float32)
    pad = _EP - _E
    row_r = jnp.concatenate([row, jnp.zeros((pad,), jnp.int32)]).reshape(_ROWS, 128)
    col_r = jnp.concatenate([col, jnp.zeros((pad,), jnp.int32)]).reshape(_ROWS, 128)
    ew_r = jnp.concatenate([ew, jnp.zeros((pad,), jnp.float32)]).reshape(_ROWS, 128)
    x_p = jnp.pad(x.astype(jnp.float32), ((0, _NP - _N), (0, 0)))
    batch2d = jnp.pad(batch.astype(jnp.int32), (0, _NP - _N),
                      constant_values=_G).reshape(1, _NP)

    degp = _deg_sc(col_r, ew_r)
    d0 = degp[:_NP].reshape(_NP, 1)
    d1 = degp[_NP:].reshape(_NP, 1)

    dinv, dinv2, h1 = _tc_a(d0, d1, x_p, W1)

    out1, norm_r = _spmm1_sc(h1, row_r, col_r, ew_r, dinv.reshape(_NP))
    h2 = _tc_layer(out1, h1, dinv2, b1.reshape(1, 16), W2, 16, 64)

    out2 = _spmm2_sc(h2, row_r, col_r, norm_r)
    h3f = _tc_layer3(out2, h2, dinv2, b2.reshape(1, 64), W3)

    out3 = _spmm3_sc(h3f, row_r, col_r, norm_r)
    g = _tc_d(out3, h3f, dinv2, b3.reshape(1, 256), batch2d,
              LW1, Lb1.reshape(1, 128), LW2, Lb2.reshape(1, 64),
              LW3.reshape(64, 1), Lb3.reshape(1, 1))
    return g.reshape(_G)


# speedup vs baseline: 10.9713x; 1.1979x over previous
import functools

import jax
import jax.numpy as jnp
from jax import lax
from jax.experimental import pallas as pl
from jax.experimental.pallas import tpu as pltpu
from jax.experimental.pallas import tpu_sc as plsc

_N = 10000
_NP = 10240
_E = 320000
_EP = 327680
_G = 128
_NSUB = 16
_NCORE = 2
_ROWS = _EP // 128
_CPT = _ROWS // 32
_CPT2 = _ROWS // 16
_SEG = _NP // _NSUB

_mesh = plsc.VectorSubcoreMesh(core_axis_name="c", subcore_axis_name="s")
_sc_params = pltpu.CompilerParams(needs_layout_passes=False,
                                  use_tc_tiling_on_sc=False)


def _splat(v16, j):
    return v16.at[jnp.full((16,), j, jnp.int32)].get(mode="promise_in_bounds")


def _scale_chunk(gb, norm_v, k, F, traced_j16=False):
    def one(j16, carry):
        nrm16 = norm_v[k, pl.ds(j16 * 16, 16)]
        for j in range(16):
            e = j16 * 16 + j
            spl = _splat(nrm16, j)
            for v in range(F // 16):
                sl = pl.ds(v * 16, 16)
                gb[e, sl] = gb[e, sl] * spl
        return carry

    if traced_j16:
        lax.fori_loop(0, 8, one, 0)
    else:
        for j16 in range(8):
            one(j16, 0)


def _pipe_spmm(h_h, acc_sh, row_v, col_v, norm_v,
               gb0, gb1, sg0, sg1, ss0, ss1, nchunks, F, traced_j16=False):
    P = nchunks // 2
    pltpu.async_copy(h_h.at[row_v.at[0]], gb0, sg0)
    pltpu.async_copy(h_h.at[row_v.at[1]], gb1, sg1)

    def body(m, carry):
        k0 = 2 * m
        k1 = 2 * m + 1
        pltpu.make_async_copy(h_h.at[row_v.at[k0]], gb0, sg0).wait()
        _scale_chunk(gb0, norm_v, k0, F, traced_j16)
        pltpu.async_copy(gb0, acc_sh.at[col_v.at[k0]], ss0, add=True)
        pltpu.make_async_copy(h_h.at[row_v.at[k1]], gb1, sg1).wait()
        _scale_chunk(gb1, norm_v, k1, F, traced_j16)
        pltpu.async_copy(gb1, acc_sh.at[col_v.at[k1]], ss1, add=True)

        @pl.when(m < P - 1)
        def _():
            pltpu.make_async_copy(gb0, acc_sh.at[col_v.at[k0]], ss0).wait()
            pltpu.async_copy(h_h.at[row_v.at[k0 + 2]], gb0, sg0)
            pltpu.make_async_copy(gb1, acc_sh.at[col_v.at[k1]], ss1).wait()
            pltpu.async_copy(h_h.at[row_v.at[k1 + 2]], gb1, sg1)

        return carry

    lax.fori_loop(0, P, body, 0)
    pltpu.make_async_copy(gb0, acc_sh.at[col_v.at[nchunks - 2]], ss0).wait()
    pltpu.make_async_copy(gb1, acc_sh.at[col_v.at[nchunks - 1]], ss1).wait()


@functools.partial(
    pl.kernel,
    out_type=jax.ShapeDtypeStruct((_NCORE * _NP,), jnp.float32),
    mesh=_mesh,
    compiler_params=_sc_params,
    scratch_types=[
        pltpu.VMEM((_CPT, 128), jnp.int32),
        pltpu.VMEM((_CPT, 128), jnp.float32),
        pltpu.VMEM((_SEG,), jnp.float32),
        pltpu.VMEM_SHARED((_NP,), jnp.float32),
    ],
)
def _deg_sc(col_h, ew_h, deg_h, col_v, ew_v, zv, acc_sh):
    c = lax.axis_index("c")
    s = lax.axis_index("s")
    w = c * _NSUB + s
    pltpu.sync_copy(col_h.at[pl.ds(w * _CPT, _CPT)], col_v)
    pltpu.sync_copy(ew_h.at[pl.ds(w * _CPT, _CPT)], ew_v)

    def zb(i, carry):
        zv[pl.ds(i * 16, 16)] = jnp.zeros((16,), jnp.float32)
        return carry

    lax.fori_loop(0, _SEG // 16, zb, 0)
    pltpu.sync_copy(zv, acc_sh.at[pl.ds(s * _SEG, _SEG)])
    plsc.subcore_barrier()

    def body(k, carry):
        pltpu.sync_copy(ew_v.at[k], acc_sh.at[col_v.at[k]], add=True)
        return carry

    lax.fori_loop(0, _CPT, body, 0)
    plsc.subcore_barrier()
    pltpu.sync_copy(
        acc_sh.at[pl.ds(s * _SEG, _SEG)],
        deg_h.at[pl.ds(c * _NP + s * _SEG, _SEG)],
    )


@functools.partial(
    pl.kernel,
    out_type=(
        jax.ShapeDtypeStruct((_NCORE * _NP, 16), jnp.float32),
        jax.ShapeDtypeStruct((_ROWS, 128), jnp.float32),
    ),
    mesh=_mesh,
    compiler_params=_sc_params,
    scratch_types=[
        pltpu.VMEM((_CPT, 128), jnp.int32),
        pltpu.VMEM((_CPT, 128), jnp.int32),
        pltpu.VMEM((_CPT, 128), jnp.float32),
        pltpu.VMEM((_CPT, 128), jnp.float32),
        pltpu.VMEM((_NP,), jnp.float32),
        pltpu.VMEM((128, 16), jnp.float32),
        pltpu.VMEM((128, 16), jnp.float32),
        pltpu.VMEM((128, 16), jnp.float32),
        pltpu.VMEM_SHARED((_NP, 16), jnp.float32),
        pltpu.SemaphoreType.DMA,
        pltpu.SemaphoreType.DMA,
        pltpu.SemaphoreType.DMA,
        pltpu.SemaphoreType.DMA,
    ],
)
def _spmm1_sc(h_h, row_h, col_h, ew_h, dinv_h, out_h, norm_out_h,
              row_v, col_v, ew_v, norm_v, dinv_v, gb0, gb1, zbuf, acc_sh,
              sg0, sg1, ss0, ss1):
    c = lax.axis_index("c")
    s = lax.axis_index("s")
    w = c * _NSUB + s
    pltpu.sync_copy(row_h.at[pl.ds(w * _CPT, _CPT)], row_v)
    pltpu.sync_copy(col_h.at[pl.ds(w * _CPT, _CPT)], col_v)
    pltpu.sync_copy(ew_h.at[pl.ds(w * _CPT, _CPT)], ew_v)
    pltpu.sync_copy(dinv_h, dinv_v)

    def zb(i, carry):
        zbuf[i, :] = jnp.zeros((16,), jnp.float32)
        return carry

    lax.fori_loop(0, 128, zb, 0)

    def za(i, carry):
        pltpu.sync_copy(zbuf, acc_sh.at[pl.ds(s * _SEG + i * 128, 128)])
        return carry

    lax.fori_loop(0, _SEG // 128, za, 0)

    def nb(k, carry):
        for j16 in range(8):
            sl = pl.ds(j16 * 16, 16)
            r16 = row_v[k, sl]
            c16 = col_v[k, sl]
            nrm = (plsc.load_gather(dinv_v, [r16]) * ew_v[k, sl]
                   * plsc.load_gather(dinv_v, [c16]))
            norm_v[k, sl] = nrm
        return carry

    lax.fori_loop(0, _CPT, nb, 0)
    pltpu.sync_copy(norm_v, norm_out_h.at[pl.ds(w * _CPT, _CPT)])
    plsc.subcore_barrier()
    _pipe_spmm(h_h, acc_sh, row_v, col_v, norm_v,
               gb0, gb1, sg0, sg1, ss0, ss1, _CPT, 16)
    plsc.subcore_barrier()
    pltpu.sync_copy(
        acc_sh.at[pl.ds(s * _SEG, _SEG)],
        out_h.at[pl.ds(c * _NP + s * _SEG, _SEG)],
    )


@functools.partial(
    pl.kernel,
    out_type=jax.ShapeDtypeStruct((_NCORE * _NP, 64), jnp.float32),
    mesh=_mesh,
    compiler_params=_sc_params,
    scratch_types=[
        pltpu.VMEM((_CPT, 128), jnp.int32),
        pltpu.VMEM((_CPT, 128), jnp.int32),
        pltpu.VMEM((_CPT, 128), jnp.float32),
        pltpu.VMEM((128, 64), jnp.float32),
        pltpu.VMEM((128, 64), jnp.float32),
        pltpu.VMEM((128, 64), jnp.float32),
        pltpu.VMEM_SHARED((_NP, 64), jnp.float32),
        pltpu.SemaphoreType.DMA,
        pltpu.SemaphoreType.DMA,
        pltpu.SemaphoreType.DMA,
        pltpu.SemaphoreType.DMA,
    ],
)
def _spmm2_sc(h_h, row_h, col_h, norm_h, out_h,
              row_v, col_v, norm_v, gb0, gb1, zbuf, acc_sh,
              sg0, sg1, ss0, ss1):
    c = lax.axis_index("c")
    s = lax.axis_index("s")
    w = c * _NSUB + s
    pltpu.sync_copy(row_h.at[pl.ds(w * _CPT, _CPT)], row_v)
    pltpu.sync_copy(col_h.at[pl.ds(w * _CPT, _CPT)], col_v)
    pltpu.sync_copy(norm_h.at[pl.ds(w * _CPT, _CPT)], norm_v)

    def zb(i, carry):
        for v in range(4):
            zbuf[i, pl.ds(v * 16, 16)] = jnp.zeros((16,), jnp.float32)
        return carry

    lax.fori_loop(0, 128, zb, 0)

    def za(i, carry):
        pltpu.sync_copy(zbuf, acc_sh.at[pl.ds(s * _SEG + i * 128, 128)])
        return carry

    lax.fori_loop(0, _SEG // 128, za, 0)
    plsc.subcore_barrier()
    _pipe_spmm(h_h, acc_sh, row_v, col_v, norm_v,
               gb0, gb1, sg0, sg1, ss0, ss1, _CPT, 64)
    plsc.subcore_barrier()
    pltpu.sync_copy(
        acc_sh.at[pl.ds(s * _SEG, _SEG)],
        out_h.at[pl.ds(c * _NP + s * _SEG, _SEG)],
    )


@functools.partial(
    pl.kernel,
    out_type=jax.ShapeDtypeStruct((_NCORE * _NP, 128), jnp.float32),
    mesh=_mesh,
    compiler_params=_sc_params,
    scratch_types=[
        pltpu.VMEM((16, 128), jnp.int32),
        pltpu.VMEM((16, 128), jnp.int32),
        pltpu.VMEM((16, 128), jnp.float32),
        pltpu.VMEM((128, 128), jnp.float32),
        pltpu.VMEM((128, 128), jnp.float32),
        pltpu.VMEM_SHARED((_NP, 128), jnp.float32),
        pltpu.SemaphoreType.DMA,
        pltpu.SemaphoreType.DMA,
        pltpu.SemaphoreType.DMA,
        pltpu.SemaphoreType.DMA,
    ],
)
def _spmm3_sc(h_h, row_h, col_h, norm_h, out_h,
              row_v, col_v, norm_v, gb0, gb1, acc_sh,
              sg0, sg1, ss0, ss1):
    c = lax.axis_index("c")
    s = lax.axis_index("s")
    off16 = jnp.full((16,), c * _NP, jnp.int32)

    def zb(i, carry):
        for v in range(8):
            gb0[i, pl.ds(v * 16, 16)] = jnp.zeros((16,), jnp.float32)
        return carry

    lax.fori_loop(0, 128, zb, 0)

    def za(i, carry):
        pltpu.sync_copy(gb0, acc_sh.at[pl.ds(s * _SEG + i * 128, 128)])
        return carry

    lax.fori_loop(0, _SEG // 128, za, 0)
    plsc.subcore_barrier()

    def slab(m, carry):
        base = s * _CPT2 + m * 16
        pltpu.sync_copy(row_h.at[pl.ds(base, 16)], row_v)
        pltpu.sync_copy(col_h.at[pl.ds(base, 16)], col_v)
        pltpu.sync_copy(norm_h.at[pl.ds(base, 16)], norm_v)

        def ob(k, carry2):
            for j16 in range(8):
                sl = pl.ds(j16 * 16, 16)
                row_v[k, sl] = row_v[k, sl] + off16
            return carry2

        lax.fori_loop(0, 16, ob, 0)
        _pipe_spmm(h_h, acc_sh, row_v, col_v, norm_v,
                   gb0, gb1, sg0, sg1, ss0, ss1, 16, 128, traced_j16=True)
        return carry

    lax.fori_loop(0, _CPT2 // 16, slab, 0)
    plsc.subcore_barrier()
    pltpu.sync_copy(
        acc_sh.at[pl.ds(s * _SEG, _SEG)],
        out_h.at[pl.ds(c * _NP + s * _SEG, _SEG)],
    )


_BT = 1024
_NPB = _NP // _BT


def _tca_body(d0, d1, xb, w1, dinv_o, dinv2_o, h1_o):
    deg = d0[...] + d1[...] + 1.0
    dinv_o[...] = lax.rsqrt(deg)
    dinv2_o[...] = 1.0 / deg
    h1_o[...] = jnp.dot(xb[...], w1[...], preferred_element_type=jnp.float32)


def _tc_a(d0, d1, x_p, W1):
    return pl.pallas_call(
        _tca_body,
        grid=(_NPB,),
        in_specs=[
            pl.BlockSpec((_BT, 1), lambda i: (i, 0)),
            pl.BlockSpec((_BT, 1), lambda i: (i, 0)),
            pl.BlockSpec((_BT, 128), lambda i: (i, 0)),
            pl.BlockSpec((128, 16), lambda i: (0, 0)),
        ],
        out_specs=[
            pl.BlockSpec((_BT, 1), lambda i: (i, 0)),
            pl.BlockSpec((_BT, 1), lambda i: (i, 0)),
            pl.BlockSpec((_BT, 16), lambda i: (i, 0)),
        ],
        out_shape=[
            jax.ShapeDtypeStruct((_NP, 1), jnp.float32),
            jax.ShapeDtypeStruct((_NP, 1), jnp.float32),
            jax.ShapeDtypeStruct((_NP, 16), jnp.float32),
        ],
    )(d0, d1, x_p, W1)


def _tcb_body(oa, ob_, hb, dinv2, bias, w, out_o):
    pre = oa[...] + ob_[...] + dinv2[...] * hb[...] + bias[...]
    act = jax.nn.silu(pre)
    out_o[...] = jnp.dot(act, w[...], preferred_element_type=jnp.float32)


def _tc_layer(outp, h, dinv2, bias, w, fin, fout):
    return pl.pallas_call(
        _tcb_body,
        grid=(_NPB,),
        in_specs=[
            pl.BlockSpec((_BT, fin), lambda i: (i, 0)),
            pl.BlockSpec((_BT, fin), lambda i: (_NPB + i, 0)),
            pl.BlockSpec((_BT, fin), lambda i: (i, 0)),
            pl.BlockSpec((_BT, 1), lambda i: (i, 0)),
            pl.BlockSpec((1, fin), lambda i: (0, 0)),
            pl.BlockSpec((fin, fout), lambda i: (0, 0)),
        ],
        out_specs=pl.BlockSpec((_BT, fout), lambda i: (i, 0)),
        out_shape=jax.ShapeDtypeStruct((_NP, fout), jnp.float32),
    )(outp, outp, h, dinv2, bias, w)


def _tcc_body(oa, ob_, hb, dinv2, bias, w, out_o):
    pre = oa[...] + ob_[...] + dinv2[...] * hb[...] + bias[...]
    act = jax.nn.silu(pre)
    out_o[...] = jnp.dot(act, w[...], preferred_element_type=jnp.float32)


def _tc_layer3(outp, h2, dinv2, b2, W3):
    return pl.pallas_call(
        _tcc_body,
        grid=(_NPB, 2),
        in_specs=[
            pl.BlockSpec((_BT, 64), lambda i, j: (i, 0)),
            pl.BlockSpec((_BT, 64), lambda i, j: (_NPB + i, 0)),
            pl.BlockSpec((_BT, 64), lambda i, j: (i, 0)),
            pl.BlockSpec((_BT, 1), lambda i, j: (i, 0)),
            pl.BlockSpec((1, 64), lambda i, j: (0, 0)),
            pl.BlockSpec((64, 128), lambda i, j: (0, j)),
        ],
        out_specs=pl.BlockSpec((_BT, 128), lambda i, j: (j * _NPB + i, 0)),
        out_shape=jax.ShapeDtypeStruct((2 * _NP, 128), jnp.float32),
    )(outp, outp, h2, dinv2, b2, W3)


def _tcd_body(o3a, o3b, h3a, h3b, dinv2, b3, batchb,
              lw1, lb1, lw2, lb2, lw3, lb3, out_o, sumsA, sumsB, cnts):
    i = pl.program_id(0)

    @pl.when(i == 0)
    def _():
        sumsA[...] = jnp.zeros_like(sumsA)
        sumsB[...] = jnp.zeros_like(sumsB)
        cnts[...] = jnp.zeros_like(cnts)

    d2 = dinv2[...]
    x4a = jax.nn.silu(o3a[...] + d2 * h3a[...] + b3[..., :128])
    x4b = jax.nn.silu(o3b[...] + d2 * h3b[...] + b3[..., 128:])
    ids = lax.broadcasted_iota(jnp.int32, (_G, _BT), 0)
    P = (batchb[...] == ids).astype(jnp.float32)
    sumsA[...] += jnp.dot(P, x4a, preferred_element_type=jnp.float32)
    sumsB[...] += jnp.dot(P, x4b, preferred_element_type=jnp.float32)
    cnts[...] += jnp.sum(P, axis=1, keepdims=True)

    @pl.when(i == _NPB - 1)
    def _():
        inv = 1.0 / jnp.maximum(cnts[...], 1.0)
        g = jnp.concatenate([sumsA[...], sumsB[...]], axis=1) * inv
        g = jax.nn.silu(jnp.dot(g, lw1[...], preferred_element_type=jnp.float32) + lb1[...])
        g = jax.nn.silu(jnp.dot(g, lw2[...], preferred_element_type=jnp.float32) + lb2[...])
        out_o[...] = jnp.dot(g, lw3[...], preferred_element_type=jnp.float32) + lb3[...]


def _tc_d(out3, h3f, dinv2, b3, batch2d, LW1, Lb1, LW2, Lb2, LW3, Lb3):
    return pl.pallas_call(
        _tcd_body,
        grid=(_NPB,),
        in_specs=[
            pl.BlockSpec((_BT, 128), lambda i: (i, 0)),
            pl.BlockSpec((_BT, 128), lambda i: (_NPB + i, 0)),
            pl.BlockSpec((_BT, 128), lambda i: (i, 0)),
            pl.BlockSpec((_BT, 128), lambda i: (_NPB + i, 0)),
            pl.BlockSpec((_BT, 1), lambda i: (i, 0)),
            pl.BlockSpec((1, 256), lambda i: (0, 0)),
            pl.BlockSpec((1, _BT), lambda i: (0, i)),
            pl.BlockSpec((256, 128), lambda i: (0, 0)),
            pl.BlockSpec((1, 128), lambda i: (0, 0)),
            pl.BlockSpec((128, 64), lambda i: (0, 0)),
            pl.BlockSpec((1, 64), lambda i: (0, 0)),
            pl.BlockSpec((64, 1), lambda i: (0, 0)),
            pl.BlockSpec((1, 1), lambda i: (0, 0)),
        ],
        out_specs=pl.BlockSpec((_G, 1), lambda i: (0, 0)),
        out_shape=jax.ShapeDtypeStruct((_G, 1), jnp.float32),
        scratch_shapes=[
            pltpu.VMEM((_G, 128), jnp.float32),
            pltpu.VMEM((_G, 128), jnp.float32),
            pltpu.VMEM((_G, 1), jnp.float32),
        ],
    )(out3, out3, h3f, h3f, dinv2, b3, batch2d,
      LW1, Lb1, LW2, Lb2, LW3, Lb3)


def kernel(x, edge_index, edge_attr, batch, W1, b1, W2, b2, W3, b3,
           LW1, Lb1, LW2, Lb2, LW3, Lb3):
    row = edge_index[0].astype(jnp.int32)
    col = edge_index[1].astype(jnp.int32)
    ew = edge_attr.astype(jnp.float32)
    pad = _EP - _E
    row_r = jnp.concatenate([row, jnp.zeros((pad,), jnp.int32)]).reshape(_ROWS, 128)
    col_r = jnp.concatenate([col, jnp.zeros((pad,), jnp.int32)]).reshape(_ROWS, 128)
    ew_r = jnp.concatenate([ew, jnp.zeros((pad,), jnp.float32)]).reshape(_ROWS, 128)
    x_p = jnp.pad(x.astype(jnp.float32), ((0, _NP - _N), (0, 0)))
    batch2d = jnp.pad(batch.astype(jnp.int32), (0, _NP - _N),
                      constant_values=_G).reshape(1, _NP)

    degp = _deg_sc(col_r, ew_r)
    d0 = degp[:_NP].reshape(_NP, 1)
    d1 = degp[_NP:].reshape(_NP, 1)

    dinv, dinv2, h1 = _tc_a(d0, d1, x_p, W1)

    out1, norm_r = _spmm1_sc(h1, row_r, col_r, ew_r, dinv.reshape(_NP))
    h2 = _tc_layer(out1, h1, dinv2, b1.reshape(1, 16), W2, 16, 64)

    out2 = _spmm2_sc(h2, row_r, col_r, norm_r)
    h3f = _tc_layer3(out2, h2, dinv2, b2.reshape(1, 64), W3)

    out3 = _spmm3_sc(h3f, row_r, col_r, norm_r)
    g = _tc_d(out3, h3f, dinv2, b3.reshape(1, 256), batch2d,
              LW1, Lb1.reshape(1, 128), LW2, Lb2.reshape(1, 64),
              LW3.reshape(64, 1), Lb3.reshape(1, 1))
    return g.reshape(_G)
